# baseline (speedup 1.0000x reference)
import functools

import jax
import jax.numpy as jnp
from jax import lax
from jax.experimental import pallas as pl
from jax.experimental.pallas import tpu as pltpu
from jax.experimental.pallas import tpu_sc as plsc

_V = 1000000
_H = _V // 2
_D = 64
_B = 4096
_S = 200
_N = _B * _S

_G = 128
_K = 4
_NC = 2
_NS = 16
_NW = _NC * _NS

_VB = 4096
_NT = (_V + 2 * _VB - 1) // (2 * _VB)
_TR = _NT * _VB


def _t1_body(a_ref, b_ref, tail_ref, out_ref):
    i = pl.program_id(0)

    @pl.when(i < _NT - 1)
    def _main():
        a = a_ref[...]
        b = b_ref[...]
        c = jnp.concatenate([a, b], axis=0)
        i128 = jnp.eye(128, dtype=jnp.float32)
        out_ref[...] = lax.dot_general(
            c, i128, (((0,), (0,)), ((), ())),
            preferred_element_type=jnp.float32)

    @pl.when(i == _NT - 1)
    def _tail():
        out_ref[...] = tail_ref[...]


def _transpose_table(table_t, tail_pre):
    last = _V // _VB - 2
    return pl.pallas_call(
        _t1_body,
        grid=(_NT,),
        in_specs=[
            pl.BlockSpec((_D, _VB), lambda i: (0, jnp.minimum(2 * i, last))),
            pl.BlockSpec((_D, _VB),
                         lambda i: (0, jnp.minimum(2 * i + 1, last + 1))),
            pl.BlockSpec((_VB, 128), lambda i: (0, 0)),
        ],
        out_specs=pl.BlockSpec((_VB, 128), lambda i: (i, 0)),
        out_shape=jax.ShapeDtypeStruct((_TR, 128), jnp.float32),
    )(table_t, table_t, tail_pre)


def _make_gather():
    rows_w = _N // _NW
    gpw = rows_w // _G
    n_ch = gpw // _K
    mesh = plsc.VectorSubcoreMesh(core_axis_name="c", subcore_axis_name="s")

    @functools.partial(
        pl.kernel,
        mesh=mesh,
        compiler_params=pltpu.CompilerParams(use_tc_tiling_on_sc=False),
        out_type=jax.ShapeDtypeStruct((_N // _G, _G, _D), jnp.float32),
        scratch_types=[
            pltpu.VMEM((gpw, _G), jnp.int32),
            pltpu.VMEM((_K, _G, _D), jnp.float32),
            pltpu.SemaphoreType.DMA,
        ],
    )
    def emb(idx_hbm, table_hbm, out_hbm, idx_v, rows_v, gsem):
        wid = lax.axis_index("s") * _NC + lax.axis_index("c")
        gbase = wid * gpw
        pltpu.sync_copy(idx_hbm.at[pl.ds(gbase, gpw)], idx_v)

        def chunk(ci, carry):
            cps = [
                pltpu.async_copy(
                    table_hbm.at[idx_v.at[ci * _K + b]], rows_v.at[b], gsem)
                for b in range(_K)
            ]
            for cp in cps:
                cp.wait()
            pltpu.sync_copy(rows_v, out_hbm.at[pl.ds(gbase + ci * _K, _K)])
            return carry

        lax.fori_loop(0, n_ch, chunk, 0)

    return emb


def _t3_body(in_ref, out_ref):
    x = in_ref[...]
    i64 = jnp.eye(64, dtype=jnp.float32)
    dn = (((0,), (0,)), ((), ()))
    for g in range(32):
        xg = x[g * 64:(g + 1) * 64, :]
        za = lax.dot_general(xg[:, :64], i64, dn,
                             preferred_element_type=jnp.float32)
        zb = lax.dot_general(xg[:, 64:], i64, dn,
                             preferred_element_type=jnp.float32)
        z = jnp.concatenate([za, zb], axis=1)
        out_ref[0, :, g, :, :] = z.reshape(8, 8, 128)


def _relayout_out(flat):
    return pl.pallas_call(
        _t3_body,
        grid=(_S,),
        in_specs=[pl.BlockSpec((2048, 128), lambda s: (s, 0))],
        out_specs=pl.BlockSpec(
            (1, 8, _B // 128, 8, 128), lambda s: (s, 0, 0, 0, 0)),
        out_shape=jax.ShapeDtypeStruct((_S, 8, _B // 128, 8, 128),
                                       jnp.float32),
    )(flat)


def kernel(token_ids, embeddings):
    ids = token_ids.astype(jnp.int32)
    j = ids & (2 * _VB - 1)
    rows = (ids - j) + 2 * (j & (_VB - 1)) + (j >> 12)
    idx = (rows.T.reshape(_S, _B // 128, 2, 64)
           .transpose(0, 1, 3, 2).reshape(_N // _G, _G))
    tail_v = (_NT - 1) * 2 * _VB
    tail_pre = jnp.pad(embeddings[tail_v:],
                       ((0, _VB - (_V - tail_v)), (0, _D)))
    table_lin = _transpose_table(embeddings.T, tail_pre).reshape(2 * _TR, _D)
    gathered = _make_gather()(idx, table_lin)
    out5d = _relayout_out(gathered.reshape(_N * _D // 128, 128))
    return jnp.transpose(out5d, (2, 4, 0, 1, 3)).reshape(_B, _S, _D)


# speedup vs baseline: 1.8842x; 1.8842x over previous
import functools

import jax
import jax.numpy as jnp
from jax import lax
from jax.experimental import pallas as pl
from jax.experimental.pallas import tpu as pltpu
from jax.experimental.pallas import tpu_sc as plsc

_V = 1000000
_H = _V // 2
_D = 64
_B = 4096
_S = 200
_N = _B * _S

_G = 128
_K = 4
_NC = 2
_NS = 16
_NW = _NC * _NS

_VB = 4096
_NT = (_V + 2 * _VB - 1) // (2 * _VB)
_TR = _NT * _VB


def _t1_body(a_ref, b_ref, tail_ref, out_ref):
    i = pl.program_id(0)

    @pl.when(i < _NT - 1)
    def _main():
        a = a_ref[...]
        b = b_ref[...]
        c = jnp.concatenate([a, b], axis=0)
        i128 = jnp.eye(128, dtype=jnp.float32)
        out_ref[...] = lax.dot_general(
            c, i128, (((0,), (0,)), ((), ())),
            preferred_element_type=jnp.float32)

    @pl.when(i == _NT - 1)
    def _tail():
        out_ref[...] = tail_ref[...]


def _transpose_table(table_t, tail_pre):
    last = _V // _VB - 2
    return pl.pallas_call(
        _t1_body,
        grid=(_NT,),
        in_specs=[
            pl.BlockSpec((_D, _VB), lambda i: (0, jnp.minimum(2 * i, last))),
            pl.BlockSpec((_D, _VB),
                         lambda i: (0, jnp.minimum(2 * i + 1, last + 1))),
            pl.BlockSpec((_VB, 128), lambda i: (0, 0)),
        ],
        out_specs=pl.BlockSpec((_VB, 128), lambda i: (i, 0)),
        out_shape=jax.ShapeDtypeStruct((_TR, 128), jnp.float32),
    )(table_t, table_t, tail_pre)


def _make_gather():
    rows_w = _N // _NW
    gpw = rows_w // _G
    n_ch = gpw // _K
    mesh = plsc.VectorSubcoreMesh(core_axis_name="c", subcore_axis_name="s")

    @functools.partial(
        pl.kernel,
        mesh=mesh,
        compiler_params=pltpu.CompilerParams(use_tc_tiling_on_sc=False),
        out_type=jax.ShapeDtypeStruct((_S, _B, _D), jnp.float32),
        scratch_types=[
            pltpu.VMEM((gpw, _G), jnp.int32),
            pltpu.VMEM((_K * _G, _D), jnp.float32),
            pltpu.SemaphoreType.DMA,
        ],
    )
    def emb(idx_hbm, table_hbm, out_hbm, idx_v, rows_v, gsem):
        wid = lax.axis_index("s") * _NC + lax.axis_index("c")
        gbase = wid * gpw
        pltpu.sync_copy(idx_hbm.at[pl.ds(gbase, gpw)], idx_v)

        def chunk(ci, carry):
            cps = [
                pltpu.async_copy(
                    table_hbm.at[idx_v.at[ci * _K + b]],
                    rows_v.at[pl.ds(b * _G, _G)], gsem)
                for b in range(_K)
            ]
            for cp in cps:
                cp.wait()
            slot0 = (gbase + ci * _K) * _G
            pltpu.sync_copy(
                rows_v,
                out_hbm.at[slot0 // _B, pl.ds(slot0 % _B, _K * _G)])
            return carry

        lax.fori_loop(0, n_ch, chunk, 0)

    return emb


def _t3_body(in_ref, out_ref):
    x = in_ref[...]
    i64 = jnp.eye(64, dtype=jnp.float32)
    dn = (((0,), (0,)), ((), ()))
    for g in range(32):
        xg = x[g * 64:(g + 1) * 64, :]
        za = lax.dot_general(xg[:, :64], i64, dn,
                             preferred_element_type=jnp.float32)
        zb = lax.dot_general(xg[:, 64:], i64, dn,
                             preferred_element_type=jnp.float32)
        z = jnp.concatenate([za, zb], axis=1)
        out_ref[0, :, g, :, :] = z.reshape(8, 8, 128)


def _relayout_out(flat):
    return pl.pallas_call(
        _t3_body,
        grid=(_S,),
        in_specs=[pl.BlockSpec((2048, 128), lambda s: (s, 0))],
        out_specs=pl.BlockSpec(
            (1, 8, _B // 128, 8, 128), lambda s: (s, 0, 0, 0, 0)),
        out_shape=jax.ShapeDtypeStruct((_S, 8, _B // 128, 8, 128),
                                       jnp.float32),
    )(flat)


def kernel(token_ids, embeddings):
    ids = token_ids.astype(jnp.int32)
    j = ids & (2 * _VB - 1)
    rows = (ids - j) + 2 * (j & (_VB - 1)) + (j >> 12)
    idx = rows.T.reshape(_N // _G, _G)
    tail_v = (_NT - 1) * 2 * _VB
    tail_pre = jnp.pad(embeddings[tail_v:],
                       ((0, _VB - (_V - tail_v)), (0, _D)))
    table_lin = _transpose_table(embeddings.T, tail_pre).reshape(2 * _TR, _D)
    gathered = _make_gather()(idx, table_lin)
    return jnp.transpose(gathered, (1, 0, 2))


# speedup vs baseline: 2.0221x; 1.0732x over previous
import functools

import jax
import jax.numpy as jnp
from jax import lax
from jax.experimental import pallas as pl
from jax.experimental.pallas import tpu as pltpu
from jax.experimental.pallas import tpu_sc as plsc

_V = 1000000
_H = _V // 2
_D = 64
_B = 4096
_S = 200
_N = _B * _S

_G = 128
_K = 4
_NC = 2
_NS = 16
_NW = _NC * _NS

_VB = 8192
_NT = (_V + 2 * _VB - 1) // (2 * _VB)
_TR = _NT * _VB


def _t1_body(a_ref, b_ref, tail_ref, out_ref):
    i = pl.program_id(0)

    @pl.when(i < _NT - 1)
    def _main():
        a = a_ref[...]
        b = b_ref[...]
        c = jnp.concatenate([a, b], axis=0)
        i128 = jnp.eye(128, dtype=jnp.float32)
        out_ref[...] = lax.dot_general(
            c, i128, (((0,), (0,)), ((), ())),
            preferred_element_type=jnp.float32)

    @pl.when(i == _NT - 1)
    def _tail():
        out_ref[...] = tail_ref[...]


def _transpose_table(table_t, tail_pre):
    last = _V // _VB - 2
    return pl.pallas_call(
        _t1_body,
        grid=(_NT,),
        in_specs=[
            pl.BlockSpec((_D, _VB), lambda i: (0, jnp.minimum(2 * i, last))),
            pl.BlockSpec((_D, _VB),
                         lambda i: (0, jnp.minimum(2 * i + 1, last + 1))),
            pl.BlockSpec((_VB, 128), lambda i: (0, 0)),
        ],
        out_specs=pl.BlockSpec((_VB, 128), lambda i: (i, 0)),
        out_shape=jax.ShapeDtypeStruct((_TR, 128), jnp.float32),
    )(table_t, table_t, tail_pre)


def _make_gather():
    rows_w = _N // _NW
    gpw = rows_w // _G
    n_ch = gpw // _K
    mesh = plsc.VectorSubcoreMesh(core_axis_name="c", subcore_axis_name="s")

    @functools.partial(
        pl.kernel,
        mesh=mesh,
        compiler_params=pltpu.CompilerParams(use_tc_tiling_on_sc=False),
        out_type=jax.ShapeDtypeStruct((_S, _B, _D), jnp.float32),
        scratch_types=[
            pltpu.VMEM((gpw, _G), jnp.int32),
            pltpu.VMEM((2, _K * _G, _D), jnp.float32),
            pltpu.SemaphoreType.DMA,
            pltpu.SemaphoreType.DMA,
        ],
    )
    def emb(idx_hbm, table_hbm, out_hbm, idx_v, rows_v, gsem, osem):
        wid = lax.axis_index("s") * _NC + lax.axis_index("c")
        gbase = wid * gpw
        pltpu.sync_copy(idx_hbm.at[pl.ds(gbase, gpw)], idx_v)

        def fire(ci, p):
            return [
                pltpu.async_copy(
                    table_hbm.at[idx_v.at[ci * _K + b]],
                    rows_v.at[p, pl.ds(b * _G, _G)], gsem)
                for b in range(_K)
            ]

        def wb(ci, p):
            slot0 = (gbase + ci * _K) * _G
            return pltpu.make_async_copy(
                rows_v.at[p],
                out_hbm.at[slot0 // _B, pl.ds(slot0 % _B, _K * _G)], osem)

        for cp in fire(0, 0):
            cp.wait()
        wb(0, 0).start()
        fire(1, 1)

        def chunk(ci, carry):
            p = lax.rem(ci, 2)
            for b in range(_K):
                pltpu.make_async_copy(
                    table_hbm.at[idx_v.at[ci * _K + b]],
                    rows_v.at[p, pl.ds(b * _G, _G)], gsem).wait()
            wb(ci, p).start()
            wb(ci - 1, 1 - p).wait()
            fire(ci + 1, 1 - p)
            return carry

        lax.fori_loop(1, n_ch - 1, chunk, 0)

        pl_last = (n_ch - 1) % 2
        for b in range(_K):
            pltpu.make_async_copy(
                table_hbm.at[idx_v.at[(n_ch - 1) * _K + b]],
                rows_v.at[pl_last, pl.ds(b * _G, _G)], gsem).wait()
        wb(n_ch - 1, pl_last).start()
        wb(n_ch - 2, 1 - pl_last).wait()
        wb(n_ch - 1, pl_last).wait()

    return emb


def _t3_body(in_ref, out_ref):
    x = in_ref[...]
    i64 = jnp.eye(64, dtype=jnp.float32)
    dn = (((0,), (0,)), ((), ()))
    for g in range(32):
        xg = x[g * 64:(g + 1) * 64, :]
        za = lax.dot_general(xg[:, :64], i64, dn,
                             preferred_element_type=jnp.float32)
        zb = lax.dot_general(xg[:, 64:], i64, dn,
                             preferred_element_type=jnp.float32)
        z = jnp.concatenate([za, zb], axis=1)
        out_ref[0, :, g, :, :] = z.reshape(8, 8, 128)


def _relayout_out(flat):
    return pl.pallas_call(
        _t3_body,
        grid=(_S,),
        in_specs=[pl.BlockSpec((2048, 128), lambda s: (s, 0))],
        out_specs=pl.BlockSpec(
            (1, 8, _B // 128, 8, 128), lambda s: (s, 0, 0, 0, 0)),
        out_shape=jax.ShapeDtypeStruct((_S, 8, _B // 128, 8, 128),
                                       jnp.float32),
    )(flat)


def kernel(token_ids, embeddings):
    ids = token_ids.astype(jnp.int32)
    j = ids & (2 * _VB - 1)
    rows = (ids - j) + 2 * (j & (_VB - 1)) + (j >> (_VB.bit_length() - 1))
    idx = rows.T.reshape(_N // _G, _G)
    tail_v = (_NT - 1) * 2 * _VB
    tail_pre = jnp.pad(embeddings[tail_v:],
                       ((0, _VB - (_V - tail_v)), (0, _D)))
    table_lin = _transpose_table(embeddings.T, tail_pre).reshape(2 * _TR, _D)
    gathered = _make_gather()(idx, table_lin)
    return jnp.transpose(gathered, (1, 0, 2))


# speedup vs baseline: 2.1478x; 1.0622x over previous
import functools

import jax
import jax.numpy as jnp
from jax import lax
from jax.experimental import pallas as pl
from jax.experimental.pallas import tpu as pltpu
from jax.experimental.pallas import tpu_sc as plsc

_V = 1000000
_H = _V // 2
_D = 64
_B = 4096
_S = 200
_N = _B * _S

_G = 128
_K = 4
_NC = 2
_NS = 16
_NW = _NC * _NS

_VB = 8192
_NT = (_V + 2 * _VB - 1) // (2 * _VB)
_TR = _NT * _VB


def _t1_body(a_ref, b_ref, tail_ref, out_ref):
    i = pl.program_id(0)

    @pl.when(i < _NT - 1)
    def _main():
        a = a_ref[...]
        b = b_ref[...]
        c = jnp.concatenate([a, b], axis=0)
        i128 = jnp.eye(128, dtype=jnp.float32)
        out_ref[...] = lax.dot_general(
            c, i128, (((0,), (0,)), ((), ())),
            preferred_element_type=jnp.float32)

    @pl.when(i == _NT - 1)
    def _tail():
        out_ref[...] = tail_ref[...]


def _transpose_table(table_t, tail_pre):
    last = _V // _VB - 2
    return pl.pallas_call(
        _t1_body,
        grid=(_NT,),
        in_specs=[
            pl.BlockSpec((_D, _VB), lambda i: (0, jnp.minimum(2 * i, last))),
            pl.BlockSpec((_D, _VB),
                         lambda i: (0, jnp.minimum(2 * i + 1, last + 1))),
            pl.BlockSpec((_VB, 128), lambda i: (0, 0)),
        ],
        out_specs=pl.BlockSpec((_VB, 128), lambda i: (i, 0)),
        out_shape=jax.ShapeDtypeStruct((_TR, 128), jnp.float32),
    )(table_t, table_t, tail_pre)


def _make_gather():
    rows_w = _N // _NW
    gpw = rows_w // _G
    n_ch = gpw // _K
    mesh = plsc.VectorSubcoreMesh(core_axis_name="c", subcore_axis_name="s")

    @functools.partial(
        pl.kernel,
        mesh=mesh,
        compiler_params=pltpu.CompilerParams(use_tc_tiling_on_sc=False),
        out_type=jax.ShapeDtypeStruct((_S, _B, 2 * _D), jnp.float32),
        scratch_types=[
            pltpu.VMEM((gpw, _G), jnp.int32),
            pltpu.VMEM((2, _K * _G, _D), jnp.float32),
            pltpu.SemaphoreType.DMA,
            pltpu.SemaphoreType.DMA,
        ],
    )
    def emb(idx_hbm, table_hbm, out_hbm, idx_v, rows_v, gsem, osem):
        wid = lax.axis_index("s") * _NC + lax.axis_index("c")
        gbase = wid * gpw
        pltpu.sync_copy(idx_hbm.at[pl.ds(gbase, gpw)], idx_v)

        def fire(ci, p):
            return [
                pltpu.async_copy(
                    table_hbm.at[idx_v.at[ci * _K + b]],
                    rows_v.at[p, pl.ds(b * _G, _G)], gsem)
                for b in range(_K)
            ]

        def wb(ci, p):
            slot0 = (gbase + ci * _K) * _G
            return pltpu.make_async_copy(
                rows_v.at[p],
                out_hbm.at[slot0 // _B, pl.ds(slot0 % _B, _K * _G),
                           pl.ds(0, _D)], osem)

        for cp in fire(0, 0):
            cp.wait()
        wb(0, 0).start()
        fire(1, 1)

        def chunk(ci, carry):
            p = lax.rem(ci, 2)
            for b in range(_K):
                pltpu.make_async_copy(
                    table_hbm.at[idx_v.at[ci * _K + b]],
                    rows_v.at[p, pl.ds(b * _G, _G)], gsem).wait()
            wb(ci, p).start()
            wb(ci - 1, 1 - p).wait()
            fire(ci + 1, 1 - p)
            return carry

        lax.fori_loop(1, n_ch - 1, chunk, 0)

        pl_last = (n_ch - 1) % 2
        for b in range(_K):
            pltpu.make_async_copy(
                table_hbm.at[idx_v.at[(n_ch - 1) * _K + b]],
                rows_v.at[pl_last, pl.ds(b * _G, _G)], gsem).wait()
        wb(n_ch - 1, pl_last).start()
        wb(n_ch - 2, 1 - pl_last).wait()
        wb(n_ch - 1, pl_last).wait()

    return emb


def _t3_body(in_ref, out_ref):
    x = in_ref[...]
    i64 = jnp.eye(64, dtype=jnp.float32)
    dn = (((0,), (0,)), ((), ()))
    for g in range(32):
        xg = x[g * 64:(g + 1) * 64, :]
        za = lax.dot_general(xg[:, :64], i64, dn,
                             preferred_element_type=jnp.float32)
        zb = lax.dot_general(xg[:, 64:], i64, dn,
                             preferred_element_type=jnp.float32)
        z = jnp.concatenate([za, zb], axis=1)
        out_ref[0, :, g, :, :] = z.reshape(8, 8, 128)


def _relayout_out(flat):
    return pl.pallas_call(
        _t3_body,
        grid=(_S,),
        in_specs=[pl.BlockSpec((2048, 128), lambda s: (s, 0))],
        out_specs=pl.BlockSpec(
            (1, 8, _B // 128, 8, 128), lambda s: (s, 0, 0, 0, 0)),
        out_shape=jax.ShapeDtypeStruct((_S, 8, _B // 128, 8, 128),
                                       jnp.float32),
    )(flat)


def kernel(token_ids, embeddings):
    ids = token_ids.astype(jnp.int32)
    j = ids & (2 * _VB - 1)
    rows = (ids - j) + 2 * (j & (_VB - 1)) + (j >> (_VB.bit_length() - 1))
    idx = rows.T.reshape(_N // _G, _G)
    tail_v = (_NT - 1) * 2 * _VB
    tail_pre = jnp.pad(embeddings[tail_v:],
                       ((0, _VB - (_V - tail_v)), (0, _D)))
    table_lin = _transpose_table(embeddings.T, tail_pre).reshape(2 * _TR, _D)
    gathered = _make_gather()(idx, table_lin)
    return jnp.transpose(gathered[:, :, :_D], (1, 0, 2))


# speedup vs baseline: 2.1542x; 1.0030x over previous
import functools

import jax
import jax.numpy as jnp
from jax import lax
from jax.experimental import pallas as pl
from jax.experimental.pallas import tpu as pltpu
from jax.experimental.pallas import tpu_sc as plsc

_V = 1000000
_H = _V // 2
_D = 64
_B = 4096
_S = 200
_N = _B * _S

_G = 128
_K = 5
_NC = 2
_NS = 16
_NW = _NC * _NS

_VB = 8192
_NT = (_V + 2 * _VB - 1) // (2 * _VB)
_TR = _NT * _VB


def _t1_body(a_ref, b_ref, tail_ref, out_ref):
    i = pl.program_id(0)

    @pl.when(i < _NT - 1)
    def _main():
        a = a_ref[...]
        b = b_ref[...]
        c = jnp.concatenate([a, b], axis=0)
        i128 = jnp.eye(128, dtype=jnp.float32)
        out_ref[...] = lax.dot_general(
            c, i128, (((0,), (0,)), ((), ())),
            preferred_element_type=jnp.float32)

    @pl.when(i == _NT - 1)
    def _tail():
        out_ref[...] = tail_ref[...]


def _transpose_table(table_t, tail_pre):
    last = _V // _VB - 2
    return pl.pallas_call(
        _t1_body,
        grid=(_NT,),
        in_specs=[
            pl.BlockSpec((_D, _VB), lambda i: (0, jnp.minimum(2 * i, last))),
            pl.BlockSpec((_D, _VB),
                         lambda i: (0, jnp.minimum(2 * i + 1, last + 1))),
            pl.BlockSpec((_VB, 128), lambda i: (0, 0)),
        ],
        out_specs=pl.BlockSpec((_VB, 128), lambda i: (i, 0)),
        out_shape=jax.ShapeDtypeStruct((_TR, 128), jnp.float32),
    )(table_t, table_t, tail_pre)


def _make_gather():
    rows_w = _N // _NW
    gpw = rows_w // _G
    n_ch = gpw // _K
    mesh = plsc.VectorSubcoreMesh(core_axis_name="c", subcore_axis_name="s")

    @functools.partial(
        pl.kernel,
        mesh=mesh,
        compiler_params=pltpu.CompilerParams(use_tc_tiling_on_sc=False),
        out_type=jax.ShapeDtypeStruct((_S, _B, 2 * _D), jnp.float32),
        scratch_types=[
            pltpu.VMEM((gpw, _G), jnp.int32),
            pltpu.VMEM((2, _K * _G, _D), jnp.float32),
            pltpu.SemaphoreType.DMA,
            pltpu.SemaphoreType.DMA,
        ],
    )
    def emb(idx_hbm, table_hbm, out_hbm, idx_v, rows_v, gsem, osem):
        wid = lax.axis_index("s") * _NC + lax.axis_index("c")
        gbase = wid * gpw
        pltpu.sync_copy(idx_hbm.at[pl.ds(gbase, gpw)], idx_v)

        def fire(ci, p):
            return [
                pltpu.async_copy(
                    table_hbm.at[idx_v.at[ci * _K + b]],
                    rows_v.at[p, pl.ds(b * _G, _G)], gsem)
                for b in range(_K)
            ]

        def wb(ci, p):
            slot0 = (gbase + ci * _K) * _G
            return pltpu.make_async_copy(
                rows_v.at[p],
                out_hbm.at[slot0 // _B, pl.ds(slot0 % _B, _K * _G),
                           pl.ds(0, _D)], osem)

        for cp in fire(0, 0):
            cp.wait()
        wb(0, 0).start()
        fire(1, 1)

        def chunk(ci, carry):
            p = lax.rem(ci, 2)
            for b in range(_K):
                pltpu.make_async_copy(
                    table_hbm.at[idx_v.at[ci * _K + b]],
                    rows_v.at[p, pl.ds(b * _G, _G)], gsem).wait()
            wb(ci, p).start()
            wb(ci - 1, 1 - p).wait()
            fire(ci + 1, 1 - p)
            return carry

        lax.fori_loop(1, n_ch - 1, chunk, 0)

        pl_last = (n_ch - 1) % 2
        for b in range(_K):
            pltpu.make_async_copy(
                table_hbm.at[idx_v.at[(n_ch - 1) * _K + b]],
                rows_v.at[pl_last, pl.ds(b * _G, _G)], gsem).wait()
        wb(n_ch - 1, pl_last).start()
        wb(n_ch - 2, 1 - pl_last).wait()
        wb(n_ch - 1, pl_last).wait()

    return emb


def _t3_body(in_ref, out_ref):
    x = in_ref[...]
    i64 = jnp.eye(64, dtype=jnp.float32)
    dn = (((0,), (0,)), ((), ()))
    for g in range(32):
        xg = x[g * 64:(g + 1) * 64, :]
        za = lax.dot_general(xg[:, :64], i64, dn,
                             preferred_element_type=jnp.float32)
        zb = lax.dot_general(xg[:, 64:], i64, dn,
                             preferred_element_type=jnp.float32)
        z = jnp.concatenate([za, zb], axis=1)
        out_ref[0, :, g, :, :] = z.reshape(8, 8, 128)


def _relayout_out(flat):
    return pl.pallas_call(
        _t3_body,
        grid=(_S,),
        in_specs=[pl.BlockSpec((2048, 128), lambda s: (s, 0))],
        out_specs=pl.BlockSpec(
            (1, 8, _B // 128, 8, 128), lambda s: (s, 0, 0, 0, 0)),
        out_shape=jax.ShapeDtypeStruct((_S, 8, _B // 128, 8, 128),
                                       jnp.float32),
    )(flat)


def kernel(token_ids, embeddings):
    ids = token_ids.astype(jnp.int32)
    j = ids & (2 * _VB - 1)
    rows = (ids - j) + 2 * (j & (_VB - 1)) + (j >> (_VB.bit_length() - 1))
    idx = rows.T.reshape(_N // _G, _G)
    tail_v = (_NT - 1) * 2 * _VB
    tail_pre = jnp.pad(embeddings[tail_v:],
                       ((0, _VB - (_V - tail_v)), (0, _D)))
    table_lin = _transpose_table(embeddings.T, tail_pre).reshape(2 * _TR, _D)
    gathered = _make_gather()(idx, table_lin)
    return jnp.transpose(gathered[:, :, :_D], (1, 0, 2))
